# Initial kernel scaffold; baseline (speedup 1.0000x reference)
import jax
import jax.numpy as jnp
from jax.experimental import pallas as pl


def kernel(x, W):
    raise NotImplementedError("write your pallas kernel here")


# speedup vs baseline: 3.9733x; 3.9733x over previous
import functools

import jax
import jax.numpy as jnp
from jax.experimental import pallas as pl
from jax.experimental.pallas import tpu as pltpu

N_GROUPS_ = 8
GROUP_SIZE_ = 8
N_EXPERTS_ = 64
TOPK_GROUPS_ = 4
TOPK_ = 2
NEG_INF_ = float("-inf")


def _gate_kernel(x_ref, wt_ref, w_out_ref, i_out_ref):
    x = x_ref[...]
    wt = wt_ref[...]
    scores = jnp.dot(x, wt, preferred_element_type=jnp.float32)

    m = jnp.max(scores, axis=-1, keepdims=True)
    e = jnp.exp(scores - m)
    p = e / jnp.sum(e, axis=-1, keepdims=True)

    bt = p.shape[0]
    lane = jax.lax.broadcasted_iota(jnp.int32, (bt, N_EXPERTS_), 1)
    lane_group = lane // GROUP_SIZE_

    gm64 = jnp.full((bt, N_EXPERTS_), NEG_INF_, dtype=jnp.float32)
    for g in range(N_GROUPS_):
        gmax = jnp.max(
            p[:, g * GROUP_SIZE_ : (g + 1) * GROUP_SIZE_], axis=-1, keepdims=True
        )
        gm64 = jnp.where(lane_group == g, gmax, gm64)

    sel = jnp.zeros((bt, N_EXPERTS_), dtype=jnp.bool_)
    gwork = gm64
    for _ in range(TOPK_GROUPS_):
        gmax = jnp.max(gwork, axis=-1, keepdims=True)
        cand = jnp.where(gwork == gmax, lane, N_EXPERTS_)
        win_lane = jnp.min(cand, axis=-1, keepdims=True)
        win_group = win_lane // GROUP_SIZE_
        hit = lane_group == win_group
        sel = jnp.logical_or(sel, hit)
        gwork = jnp.where(hit, NEG_INF_, gwork)

    masked = jnp.where(sel, p, NEG_INF_)

    ws = []
    idxs = []
    for _ in range(TOPK_):
        vmax = jnp.max(masked, axis=-1, keepdims=True)
        cand = jnp.where(masked == vmax, lane, N_EXPERTS_)
        win = jnp.min(cand, axis=-1, keepdims=True)
        ws.append(vmax)
        idxs.append(win)
        masked = jnp.where(lane == win, NEG_INF_, masked)

    w_out_ref[...] = jnp.concatenate(ws, axis=-1)
    i_out_ref[...] = jnp.concatenate(idxs, axis=-1)


@jax.jit
def kernel(x, W):
    T, D = x.shape
    bt = 512
    wt = W.T
    grid = (T // bt,)
    weights, indices = pl.pallas_call(
        _gate_kernel,
        grid=grid,
        in_specs=[
            pl.BlockSpec((bt, D), lambda i: (i, 0)),
            pl.BlockSpec((D, N_EXPERTS_), lambda i: (0, 0)),
        ],
        out_specs=[
            pl.BlockSpec((bt, TOPK_), lambda i: (i, 0)),
            pl.BlockSpec((bt, TOPK_), lambda i: (i, 0)),
        ],
        out_shape=[
            jax.ShapeDtypeStruct((T, TOPK_), jnp.float32),
            jax.ShapeDtypeStruct((T, TOPK_), jnp.int32),
        ],
        compiler_params=pltpu.CompilerParams(
            dimension_semantics=("arbitrary",),
        ),
    )(x, wt)
    return weights.astype(x.dtype), indices


# speedup vs baseline: 8.7604x; 2.2048x over previous
import functools

import jax
import jax.numpy as jnp
from jax.experimental import pallas as pl
from jax.experimental.pallas import tpu as pltpu

N_GROUPS_ = 8
GROUP_SIZE_ = 8
N_EXPERTS_ = 64
TOPK_GROUPS_ = 4
TOPK_ = 2
NEG_INF_ = float("-inf")


def _gate_kernel(x_ref, w_ref, w_out_ref, i_out_ref):
    st = jax.lax.dot_general(
        w_ref[...],
        x_ref[...],
        (((1,), (1,)), ((), ())),
        preferred_element_type=jnp.float32,
    )
    bt = st.shape[1]

    row_max = jnp.max(st, axis=0, keepdims=True)
    e = jnp.exp(st - row_max)
    p = e / jnp.sum(e, axis=0, keepdims=True)

    gms = [
        jnp.max(p[g * GROUP_SIZE_ : (g + 1) * GROUP_SIZE_], axis=0, keepdims=True)
        for g in range(N_GROUPS_)
    ]

    sels = []
    for g in range(N_GROUPS_):
        cnt = None
        for h in range(N_GROUPS_):
            if h == g:
                continue
            if h < g:
                beats = gms[h] >= gms[g]
            else:
                beats = gms[h] > gms[g]
            b = beats.astype(jnp.int32)
            cnt = b if cnt is None else cnt + b
        sels.append(cnt < TOPK_GROUPS_)

    masked = jnp.concatenate(
        [
            jnp.where(
                sels[g], p[g * GROUP_SIZE_ : (g + 1) * GROUP_SIZE_], NEG_INF_
            )
            for g in range(N_GROUPS_)
        ],
        axis=0,
    )

    expert_id = jax.lax.broadcasted_iota(jnp.int32, (N_EXPERTS_, bt), 0)

    ws = []
    idxs = []
    for _ in range(TOPK_):
        vmax = jnp.max(masked, axis=0, keepdims=True)
        cand = jnp.where(masked == vmax, expert_id, N_EXPERTS_)
        win = jnp.min(cand, axis=0, keepdims=True)
        ws.append(vmax)
        idxs.append(win)
        masked = jnp.where(expert_id == win, NEG_INF_, masked)

    w_out_ref[...] = jnp.concatenate(ws, axis=0)
    i_out_ref[...] = jnp.concatenate(idxs, axis=0)


@jax.jit
def kernel(x, W):
    T, D = x.shape
    bt = 512
    grid = (T // bt,)
    weights_t, indices_t = pl.pallas_call(
        _gate_kernel,
        grid=grid,
        in_specs=[
            pl.BlockSpec((bt, D), lambda i: (i, 0)),
            pl.BlockSpec((N_EXPERTS_, D), lambda i: (0, 0)),
        ],
        out_specs=[
            pl.BlockSpec((TOPK_, bt), lambda i: (0, i)),
            pl.BlockSpec((TOPK_, bt), lambda i: (0, i)),
        ],
        out_shape=[
            jax.ShapeDtypeStruct((TOPK_, T), jnp.float32),
            jax.ShapeDtypeStruct((TOPK_, T), jnp.int32),
        ],
        compiler_params=pltpu.CompilerParams(
            dimension_semantics=("arbitrary",),
        ),
    )(x, W)
    return weights_t.T.astype(x.dtype), indices_t.T


# speedup vs baseline: 10.1152x; 1.1546x over previous
import functools

import jax
import jax.numpy as jnp
from jax.experimental import pallas as pl
from jax.experimental.pallas import tpu as pltpu

N_GROUPS_ = 8
GROUP_SIZE_ = 8
N_EXPERTS_ = 64
TOPK_GROUPS_ = 4
TOPK_ = 2
NEG_INF_ = float("-inf")


def _gate_kernel(x_ref, w_ref, w_out_ref, i_out_ref):
    st = jax.lax.dot_general(
        w_ref[...],
        x_ref[...],
        (((1,), (1,)), ((), ())),
        preferred_element_type=jnp.float32,
    )
    bt = st.shape[1]

    row_max = jnp.max(st, axis=0, keepdims=True)
    e = jnp.exp(st - row_max)
    p = e / jnp.sum(e, axis=0, keepdims=True)

    gms = [
        jnp.max(p[g * GROUP_SIZE_ : (g + 1) * GROUP_SIZE_], axis=0, keepdims=True)
        for g in range(N_GROUPS_)
    ]

    sels = []
    for g in range(N_GROUPS_):
        cnt = None
        for h in range(N_GROUPS_):
            if h == g:
                continue
            if h < g:
                beats = gms[h] >= gms[g]
            else:
                beats = gms[h] > gms[g]
            b = beats.astype(jnp.int32)
            cnt = b if cnt is None else cnt + b
        sels.append(cnt < TOPK_GROUPS_)

    masked = jnp.concatenate(
        [
            jnp.where(
                sels[g], p[g * GROUP_SIZE_ : (g + 1) * GROUP_SIZE_], NEG_INF_
            )
            for g in range(N_GROUPS_)
        ],
        axis=0,
    )

    expert_id = jax.lax.broadcasted_iota(jnp.int32, (N_EXPERTS_, bt), 0)

    ws = []
    idxs = []
    for _ in range(TOPK_):
        vmax = jnp.max(masked, axis=0, keepdims=True)
        cand = jnp.where(masked == vmax, expert_id, N_EXPERTS_)
        win = jnp.min(cand, axis=0, keepdims=True)
        ws.append(vmax)
        idxs.append(win)
        masked = jnp.where(expert_id == win, NEG_INF_, masked)

    w_out_ref[...] = jnp.concatenate(ws, axis=0)
    i_out_ref[...] = jnp.concatenate(idxs, axis=0)


@jax.jit
def kernel(x, W):
    T, D = x.shape
    bt = 1024
    grid = (T // bt,)
    weights_t, indices_t = pl.pallas_call(
        _gate_kernel,
        grid=grid,
        in_specs=[
            pl.BlockSpec((bt, D), lambda i: (i, 0)),
            pl.BlockSpec((N_EXPERTS_, D), lambda i: (0, 0)),
        ],
        out_specs=[
            pl.BlockSpec((TOPK_, bt), lambda i: (0, i)),
            pl.BlockSpec((TOPK_, bt), lambda i: (0, i)),
        ],
        out_shape=[
            jax.ShapeDtypeStruct((TOPK_, T), jnp.float32),
            jax.ShapeDtypeStruct((TOPK_, T), jnp.int32),
        ],
        compiler_params=pltpu.CompilerParams(
            dimension_semantics=("arbitrary",),
        ),
    )(x, W)
    return weights_t.T.astype(x.dtype), indices_t.T
